# baseline (speedup 1.0000x reference)
import jax
import jax.numpy as jnp
from jax import lax
from jax.experimental import pallas as pl
from jax.experimental.pallas import tpu as pltpu
from jax.experimental.pallas import tpu_sc as plsc

N_NODES = 100000
N_PAD = 100096
ROWS_PER_TILE_N = N_PAD // 16
E_ROWS = 50000
Q_ROWS = 1568
K_BY_FEAT = {8: 8, 16: 4}


def _edge_loop(table_hbm, idx_hbm, dummy_hbm, shared, src_v, dst_v, rows_v,
               gsem, isem, ssems, row_base, n_chunks):
  K = src_v.shape[1]

  def fire_idx(g, b):
    rows = pl.ds(row_base + g * K, K)
    pltpu.async_copy(idx_hbm.at[0, rows], src_v.at[b], isem)
    pltpu.async_copy(idx_hbm.at[1, rows], dst_v.at[b], isem)

  def stage(g, b):
    bp = (b + 2) % 3
    bn = (b + 1) % 3

    @pl.when((g >= 1) & (g <= n_chunks))
    def _():
      pltpu.make_async_copy(dummy_hbm, rows_v.at[bp], gsem).wait()
      for j in range(K):
        pltpu.async_copy(rows_v.at[bp, j], shared.at[dst_v.at[bp, j]],
                         ssems[bp], add=True)

    @pl.when((g >= 2) & (g <= n_chunks + 1))
    def _():
      pltpu.make_async_copy(dummy_hbm, rows_v.at[bn], ssems[bn]).wait()

    @pl.when(g <= n_chunks - 1)
    def _():
      rows = pl.ds(row_base + g * K, K)
      pltpu.make_async_copy(idx_hbm.at[0, rows], src_v.at[b], isem).wait()
      pltpu.make_async_copy(idx_hbm.at[1, rows], dst_v.at[b], isem).wait()
      for j in range(K):
        pltpu.async_copy(table_hbm.at[src_v.at[b, j]], rows_v.at[b, j],
                         gsem)

    @pl.when(g <= n_chunks - 2)
    def _():
      fire_idx(g + 1, bn)

  def outer(go, carry):
    for p in range(3):
      stage(go * 3 + p, p)
    return carry

  fire_idx(0, 0)
  lax.fori_loop(0, (n_chunks + 2 + 2) // 3, outer, 0)


def _sc_agg(table, idx3d, zeros, feat):
  mesh = plsc.VectorSubcoreMesh(core_axis_name="c", subcore_axis_name="s")

  def body(table_hbm, idx_hbm, zeros_hbm, dummy_hbm, out0, out1,
           src_v, dst_v, rows_v, shared, gsem, isem, ssem0, ssem1, ssem2):
    cid = lax.axis_index("c")
    sid = lax.axis_index("s")
    nbase = sid * ROWS_PER_TILE_N
    nslice = pl.ds(nbase, ROWS_PER_TILE_N)
    pltpu.sync_copy(zeros_hbm.at[nslice], shared.at[nslice])
    plsc.subcore_barrier()
    row_base = cid * (E_ROWS // 2) + sid * Q_ROWS
    n_rows = jnp.where(sid < 15, Q_ROWS, E_ROWS // 2 - 15 * Q_ROWS)
    _edge_loop(table_hbm, idx_hbm, dummy_hbm, shared, src_v, dst_v, rows_v,
               gsem, isem, (ssem0, ssem1, ssem2), row_base,
               n_rows // K_BY_FEAT[feat])
    plsc.subcore_barrier()

    @pl.when(cid == 0)
    def _():
      pltpu.sync_copy(shared.at[nslice], out0.at[nslice])

    @pl.when(cid == 1)
    def _():
      pltpu.sync_copy(shared.at[nslice], out1.at[nslice])

  out_t = jax.ShapeDtypeStruct((N_PAD, feat), jnp.float32)
  return pl.kernel(
      body,
      out_type=(out_t, out_t),
      mesh=mesh,
      compiler_params=pltpu.CompilerParams(use_tc_tiling_on_sc=False),
      scratch_types=[
          pltpu.VMEM((3, K_BY_FEAT[feat], 128), jnp.int32),
          pltpu.VMEM((3, K_BY_FEAT[feat], 128), jnp.int32),
          pltpu.VMEM((3, K_BY_FEAT[feat], 128, feat), jnp.float32),
          pltpu.VMEM_SHARED((N_PAD, feat), jnp.float32),
          pltpu.SemaphoreType.DMA,
          pltpu.SemaphoreType.DMA,
          pltpu.SemaphoreType.DMA,
          pltpu.SemaphoreType.DMA,
          pltpu.SemaphoreType.DMA,
      ],
  )(table, idx3d, zeros,
    jnp.zeros((K_BY_FEAT[feat], 128, feat), jnp.float32))


def _tc_mlp(parts, wa, ba, wb, bb, fout):
  bm = 2048
  grid = (N_PAD + bm - 1) // bm
  fin = parts[0].shape[1]
  fmid = wa.shape[1]

  def body(*refs):
    part_refs, (wa_ref, ba_ref, wb_ref, bb_ref), (o_ref,) = (
        refs[:len(parts)], refs[len(parts):-1], refs[-1:])
    h = part_refs[0][...]
    for r in part_refs[1:]:
      h = h + r[...]
    h = jnp.dot(h, wa_ref[...], preferred_element_type=jnp.float32)
    h = jnp.maximum(h + ba_ref[...], 0.0)
    o_ref[...] = (jnp.dot(h, wb_ref[...], preferred_element_type=jnp.float32)
                  + bb_ref[...])

  node_spec = pl.BlockSpec((bm, fin), lambda i: (i, 0))
  return pl.pallas_call(
      body,
      grid=(grid,),
      in_specs=[node_spec] * len(parts) + [
          pl.BlockSpec((fin, fmid), lambda i: (0, 0)),
          pl.BlockSpec((1, fmid), lambda i: (0, 0)),
          pl.BlockSpec((fmid, fout), lambda i: (0, 0)),
          pl.BlockSpec((1, fout), lambda i: (0, 0)),
      ],
      out_specs=pl.BlockSpec((bm, fout), lambda i: (i, 0)),
      out_shape=jax.ShapeDtypeStruct((N_PAD, fout), jnp.float32),
  )(*parts, wa, ba, wb, bb)


@jax.jit
def kernel(x, edge_index, W1a, b1a, W1b, b1b, W2a, b2a, W2b, b2b):
  idx3d = edge_index.astype(jnp.int32).reshape(2, E_ROWS, 128)

  xp = jnp.pad(x, ((0, N_PAD - N_NODES), (0, 3)))
  zeros8 = jnp.zeros((N_PAD, 8), jnp.float32)
  zeros16 = jnp.zeros((N_PAD, 16), jnp.float32)
  W1a_p = jnp.pad(W1a, ((0, 3), (0, 0)))

  p0, p1 = _sc_agg(xp, idx3d, zeros8, 8)
  h1 = _tc_mlp((xp, p0, p1), W1a_p, b1a.reshape(1, -1), W1b,
               b1b.reshape(1, -1), 16)
  q0, q1 = _sc_agg(h1, idx3d, zeros16, 16)
  out = _tc_mlp((h1, q0, q1), W2a, b2a.reshape(1, -1), W2b,
                b2b.reshape(1, -1), 2)
  return out[:N_NODES]


# speedup vs baseline: 1.0104x; 1.0104x over previous
import jax
import jax.numpy as jnp
from jax import lax
from jax.experimental import pallas as pl
from jax.experimental.pallas import tpu as pltpu
from jax.experimental.pallas import tpu_sc as plsc

N_NODES = 100000
N_PAD = 100096
ROWS_PER_TILE_N = N_PAD // 16
E_ROWS = 50000
Q_ROWS = 1568
K_BY_FEAT = {8: 8, 16: 4}


def _edge_loop(table_hbm, idx_hbm, dummy_hbm, shared, src_v, dst_v, rows_v,
               gsem, isem, ssems, row_base, n_chunks):
  K = src_v.shape[1]

  def fire_idx(g, b):
    rows = pl.ds(row_base + g * K, K)
    pltpu.async_copy(idx_hbm.at[0, rows], src_v.at[b], isem)
    pltpu.async_copy(idx_hbm.at[1, rows], dst_v.at[b], isem)

  def stage(g, b):
    bp = (b + 2) % 3
    bn = (b + 1) % 3

    @pl.when((g >= 1) & (g <= n_chunks))
    def _():
      pltpu.make_async_copy(dummy_hbm, rows_v.at[bp], gsem).wait()
      for j in range(K):
        pltpu.async_copy(rows_v.at[bp, j], shared.at[dst_v.at[bp, j]],
                         ssems[bp], add=True)

    @pl.when((g >= 2) & (g <= n_chunks + 1))
    def _():
      pltpu.make_async_copy(dummy_hbm, rows_v.at[bn], ssems[bn]).wait()

    @pl.when(g <= n_chunks - 1)
    def _():
      rows = pl.ds(row_base + g * K, K)
      pltpu.make_async_copy(idx_hbm.at[0, rows], src_v.at[b], isem).wait()
      pltpu.make_async_copy(idx_hbm.at[1, rows], dst_v.at[b], isem).wait()
      for j in range(K):
        pltpu.async_copy(table_hbm.at[src_v.at[b, j]], rows_v.at[b, j],
                         gsem)

    @pl.when(g <= n_chunks - 2)
    def _():
      fire_idx(g + 1, bn)

  def outer(go, carry):
    for p in range(3):
      stage(go * 3 + p, p)
    return carry

  fire_idx(0, 0)
  lax.fori_loop(0, (n_chunks + 2 + 2) // 3, outer, 0)


def _sc_agg(table, idx3d, zeros, feat):
  mesh = plsc.VectorSubcoreMesh(core_axis_name="c", subcore_axis_name="s")

  def body(table_hbm, idx_hbm, zeros_hbm, dummy_hbm, out0, out1,
           src_v, dst_v, rows_v, shared, gsem, isem, ssem0, ssem1, ssem2):
    cid = lax.axis_index("c")
    sid = lax.axis_index("s")
    nbase = sid * ROWS_PER_TILE_N
    nslice = pl.ds(nbase, ROWS_PER_TILE_N)

    @pl.when(cid == 0)
    def _():
      pltpu.sync_copy(table_hbm.at[nslice], shared.at[nslice])

    @pl.when(cid == 1)
    def _():
      pltpu.sync_copy(zeros_hbm.at[nslice], shared.at[nslice])

    plsc.subcore_barrier()
    row_base = cid * (E_ROWS // 2) + sid * Q_ROWS
    n_rows = jnp.where(sid < 15, Q_ROWS, E_ROWS // 2 - 15 * Q_ROWS)
    _edge_loop(table_hbm, idx_hbm, dummy_hbm, shared, src_v, dst_v, rows_v,
               gsem, isem, (ssem0, ssem1, ssem2), row_base,
               n_rows // K_BY_FEAT[feat])
    plsc.subcore_barrier()

    @pl.when(cid == 0)
    def _():
      pltpu.sync_copy(shared.at[nslice], out0.at[nslice])

    @pl.when(cid == 1)
    def _():
      pltpu.sync_copy(shared.at[nslice], out1.at[nslice])

  out_t = jax.ShapeDtypeStruct((N_PAD, feat), jnp.float32)
  return pl.kernel(
      body,
      out_type=(out_t, out_t),
      mesh=mesh,
      compiler_params=pltpu.CompilerParams(use_tc_tiling_on_sc=False),
      scratch_types=[
          pltpu.VMEM((3, K_BY_FEAT[feat], 128), jnp.int32),
          pltpu.VMEM((3, K_BY_FEAT[feat], 128), jnp.int32),
          pltpu.VMEM((3, K_BY_FEAT[feat], 128, feat), jnp.float32),
          pltpu.VMEM_SHARED((N_PAD, feat), jnp.float32),
          pltpu.SemaphoreType.DMA,
          pltpu.SemaphoreType.DMA,
          pltpu.SemaphoreType.DMA,
          pltpu.SemaphoreType.DMA,
          pltpu.SemaphoreType.DMA,
      ],
  )(table, idx3d, zeros,
    jnp.zeros((K_BY_FEAT[feat], 128, feat), jnp.float32))


def _tc_mlp(parts, wa, ba, wb, bb, fout):
  bm = 2048
  grid = (N_PAD + bm - 1) // bm
  fin = parts[0].shape[1]
  fmid = wa.shape[1]

  def body(*refs):
    part_refs, (wa_ref, ba_ref, wb_ref, bb_ref), (o_ref,) = (
        refs[:len(parts)], refs[len(parts):-1], refs[-1:])
    h = part_refs[0][...]
    for r in part_refs[1:]:
      h = h + r[...]
    h = jnp.dot(h, wa_ref[...], preferred_element_type=jnp.float32)
    h = jnp.maximum(h + ba_ref[...], 0.0)
    o_ref[...] = (jnp.dot(h, wb_ref[...], preferred_element_type=jnp.float32)
                  + bb_ref[...])

  node_spec = pl.BlockSpec((bm, fin), lambda i: (i, 0))
  return pl.pallas_call(
      body,
      grid=(grid,),
      in_specs=[node_spec] * len(parts) + [
          pl.BlockSpec((fin, fmid), lambda i: (0, 0)),
          pl.BlockSpec((1, fmid), lambda i: (0, 0)),
          pl.BlockSpec((fmid, fout), lambda i: (0, 0)),
          pl.BlockSpec((1, fout), lambda i: (0, 0)),
      ],
      out_specs=pl.BlockSpec((bm, fout), lambda i: (i, 0)),
      out_shape=jax.ShapeDtypeStruct((N_PAD, fout), jnp.float32),
  )(*parts, wa, ba, wb, bb)


@jax.jit
def kernel(x, edge_index, W1a, b1a, W1b, b1b, W2a, b2a, W2b, b2b):
  idx3d = edge_index.astype(jnp.int32).reshape(2, E_ROWS, 128)

  xp = jnp.pad(x, ((0, N_PAD - N_NODES), (0, 3)))
  zeros8 = jnp.zeros((N_PAD, 8), jnp.float32)
  zeros16 = jnp.zeros((N_PAD, 16), jnp.float32)
  W1a_p = jnp.pad(W1a, ((0, 3), (0, 0)))

  p0, p1 = _sc_agg(xp, idx3d, zeros8, 8)
  h1 = _tc_mlp((p0, p1), W1a_p, b1a.reshape(1, -1), W1b,
               b1b.reshape(1, -1), 16)
  q0, q1 = _sc_agg(h1, idx3d, zeros16, 16)
  out = _tc_mlp((q0, q1), W2a, b2a.reshape(1, -1), W2b,
                b2b.reshape(1, -1), 2)
  return out[:N_NODES]
